# baseline (speedup 1.0000x reference)
import functools

import jax
import jax.numpy as jnp
from jax import lax
from jax.experimental import pallas as pl
from jax.experimental.pallas import tpu as pltpu
from jax.experimental.pallas import tpu_sc as plsc

N = 25000
D = 64
E = 800000
NC, NS = 2, 16
CH = 80
TPB = 25
BLOCKS = 25
EPT = BLOCKS * TPB * CH
N_PAD = 25088
STRIPE = N_PAD // NS
CW = 16


def _mesh():
    return plsc.VectorSubcoreMesh(core_axis_name="c", subcore_axis_name="s",
                                  num_cores=NC, num_subcores=NS)


NBUF = 3
NFB = 2


def _sc_agg_body(eidx, xall, z64, s_all, idx_g, idx_s,
                 g0, g1, g2, f0, f1, acc_sh, sg0, sg1, sg2, ss0, ss1,
                 sem_ig, sem_is):
    gbufs = (g0, g1, g2)
    fbufs = (f0, f1)
    sgs = (sg0, sg1, sg2)
    sss = (ss0, ss1)
    c = lax.axis_index("c")
    s = lax.axis_index("s")
    row0 = s * STRIPE

    pltpu.sync_copy(z64, acc_sh.at[pl.ds(row0, STRIPE)])
    plsc.subcore_barrier()

    def fire_gather(j, par):
        pltpu.async_copy(xall.at[idx_g.at[par, j]], gbufs[j % NBUF],
                         sgs[j % NBUF])

    def wait_gather(j):
        pltpu.make_async_copy(xall.at[pl.ds(0, CH)], gbufs[j % NBUF],
                              sgs[j % NBUF]).wait()

    def convert(j):
        gb = gbufs[j % NBUF]
        fb = fbufs[j % NFB]

        @pl.loop(0, CH, step=8)
        def _rows(r):
            for rr in range(8):
                for k in range(2):
                    v = gb[r + rr, pl.ds(32 * k, 32)]
                    ev, od = plsc.unpack(v, format=plsc.PackFormat.INTERLEAVED)
                    fb[r + rr, pl.ds(16 * k, 16)] = ev
                    fb[r + rr, pl.ds(32 + 16 * k, 16)] = od

    pltpu.async_copy(eidx.at[c, s, 0], idx_g.at[0], sem_ig)
    pltpu.async_copy(eidx.at[2 + c, s, 0], idx_s.at[0], sem_is)

    @pl.loop(0, BLOCKS)
    def _block(b):
        par = lax.rem(b, 2)
        pltpu.make_async_copy(eidx.at[c, s, b], idx_g.at[par],
                              sem_ig).wait()
        pltpu.make_async_copy(eidx.at[2 + c, s, b], idx_s.at[par],
                              sem_is).wait()

        @pl.when(b + 1 < BLOCKS)
        def _():
            npar = lax.rem(b + 1, 2)
            pltpu.async_copy(eidx.at[c, s, b + 1], idx_g.at[npar], sem_ig)
            pltpu.async_copy(eidx.at[2 + c, s, b + 1], idx_s.at[npar],
                             sem_is)

        fire_gather(0, par)
        fire_gather(1, par)
        scat = [None] * TPB
        for j in range(TPB):
            if j + 2 < TPB:
                fire_gather(j + 2, par)
            wait_gather(j)
            if j >= NFB:
                scat[j - NFB].wait()
            convert(j)
            scat[j] = pltpu.async_copy(fbufs[j % NFB],
                                       acc_sh.at[idx_s.at[par, j]],
                                       sss[j % NFB], add=True)
        scat[TPB - 2].wait()
        scat[TPB - 1].wait()

    plsc.subcore_barrier()
    pltpu.sync_copy(acc_sh.at[pl.ds(row0, STRIPE)],
                    s_all.at[c, pl.ds(row0, STRIPE)])


def _sc_counts_body(eidx, z16, ones_h, cnt_all, idx_s, ones_v, cnt_sh,
                    sem_cs):
    c = lax.axis_index("c")
    s = lax.axis_index("s")
    row0 = s * STRIPE

    pltpu.sync_copy(z16, cnt_sh.at[pl.ds(row0, STRIPE)])
    pltpu.sync_copy(ones_h, ones_v)
    plsc.subcore_barrier()

    @pl.loop(0, BLOCKS)
    def _block(b):
        pltpu.sync_copy(eidx.at[2 + c, s, b], idx_s)
        scat = []
        for j in range(TPB):
            scat.append(pltpu.async_copy(ones_v, cnt_sh.at[idx_s.at[j]],
                                         sem_cs, add=True))
        for h in scat:
            h.wait()

    plsc.subcore_barrier()
    pltpu.sync_copy(cnt_sh.at[pl.ds(row0, STRIPE)],
                    cnt_all.at[c, pl.ds(row0, STRIPE)])


def _make_sc_agg():
    out = jax.ShapeDtypeStruct((NC, N_PAD, D), jnp.float32)
    scratch = [
        pltpu.VMEM((2, TPB, CH), jnp.int32),
        pltpu.VMEM((2, TPB, CH), jnp.int32),
    ] + [pltpu.VMEM((CH, D), jnp.bfloat16)] * NBUF \
      + [pltpu.VMEM((CH, D), jnp.float32)] * NFB + [
        pltpu.VMEM_SHARED((N_PAD, D), jnp.float32),
    ] + [pltpu.SemaphoreType.DMA] * (NBUF + NFB + 2)
    return pl.kernel(_sc_agg_body, out_type=out, mesh=_mesh(),
                     scratch_types=scratch,
                     compiler_params=pltpu.CompilerParams(
                         use_tc_tiling_on_sc=False,
                         needs_layout_passes=False))


def _make_sc_counts():
    out = jax.ShapeDtypeStruct((NC, N_PAD, CW), jnp.float32)
    scratch = [
        pltpu.VMEM((TPB, CH), jnp.int32),
        pltpu.VMEM((CH, CW), jnp.float32),
        pltpu.VMEM_SHARED((N_PAD, CW), jnp.float32),
        pltpu.SemaphoreType.DMA,
    ]
    return pl.kernel(_sc_counts_body, out_type=out, mesh=_mesh(),
                     scratch_types=scratch,
                     compiler_params=pltpu.CompilerParams(
                         use_tc_tiling_on_sc=False))


def _tc_layer_body(s_ref, c_ref, x_ref, wl_ref, wr_ref, b_ref,
                   o_ref, ot_ref):
    cnt = jnp.maximum(c_ref[0, :, 0:1], 1.0)
    mean = s_ref[0] / cnt
    acc = jnp.dot(mean, wl_ref[0], preferred_element_type=jnp.float32)
    acc += jnp.dot(x_ref[0], wr_ref[0], preferred_element_type=jnp.float32)
    o = jnp.maximum(acc + b_ref[0], 0.0)
    o_ref[0] = o
    ot_ref[0] = o.astype(jnp.bfloat16)


def _tc_final_body(s_ref, c_ref, x1_ref, x0_ref, wl_ref, wr_ref, b_ref,
                   wt_ref, bt_ref, o_ref):
    cnt = jnp.maximum(c_ref[0, :, 0:1], 1.0)
    mean = s_ref[0] / cnt
    acc = jnp.dot(mean, wl_ref[0], preferred_element_type=jnp.float32)
    acc += jnp.dot(x1_ref[0], wr_ref[0], preferred_element_type=jnp.float32)
    x2 = jnp.maximum(acc + b_ref[0], 0.0)
    m = (x0_ref[0] + x1_ref[0] + x2) * (1.0 / 3.0)
    o_ref[0] = jnp.dot(m, wt_ref[0],
                       preferred_element_type=jnp.float32) + bt_ref[0]


_RB = 1568


def _row_spec(w):
    return pl.BlockSpec((1, _RB, w), lambda d, i: (d, i, 0))


def _w_spec(r, w):
    return pl.BlockSpec((1, r, w), lambda d, i: (d, 0, 0))


def _tc_layer(S, cnt, x, wl, wr, b):
    return pl.pallas_call(
        _tc_layer_body,
        grid=(NC, N_PAD // _RB),
        in_specs=[_row_spec(D), _row_spec(CW), _row_spec(D),
                  _w_spec(D, D), _w_spec(D, D), _w_spec(1, D)],
        out_specs=[_row_spec(D),
                   pl.BlockSpec((1, _RB, D), lambda d, i: (1 - d, i, 0))],
        out_shape=[jax.ShapeDtypeStruct((NC, N_PAD, D), jnp.float32),
                   jax.ShapeDtypeStruct((NC, N_PAD, D), jnp.bfloat16)],
    )(S, cnt, x, wl, wr, b)


def _tc_final(S, cnt, x1, x0, wl, wr, b, wt, bt):
    return pl.pallas_call(
        _tc_final_body,
        grid=(NC, N_PAD // _RB),
        in_specs=[_row_spec(D), _row_spec(CW), _row_spec(D), _row_spec(D),
                  _w_spec(D, D), _w_spec(D, D), _w_spec(1, D),
                  _w_spec(D, D), _w_spec(1, D)],
        out_specs=_row_spec(D),
        out_shape=jax.ShapeDtypeStruct((NC, N_PAD, D), jnp.float32),
    )(S, cnt, x1, x0, wl, wr, b, wt, bt)


_PERM = tuple(
    [p for p in range(0, 32, 2)] + [p for p in range(32, 64, 2)]
    + [p for p in range(1, 32, 2)] + [p for p in range(33, 64, 2)])


def _pack(x):
    return x.astype(jnp.bfloat16)


def kernel(edge_index, emb_user, emb_item, emb_cat, Wl, bl, Wr, Wt, bt):
    del emb_cat
    edge_index = edge_index.astype(jnp.int32)

    eidx = jnp.stack([edge_index[0], edge_index[1] + N_PAD,
                      edge_index[1], edge_index[0]])
    eidx = eidx.reshape(4, NS, BLOCKS, TPB, CH)

    zpad = jnp.zeros((N_PAD - N, D), jnp.float32)
    xu0 = jnp.concatenate([emb_user, zpad], axis=0)
    xi0 = jnp.concatenate([emb_item, zpad], axis=0)

    z64 = jnp.zeros((STRIPE, D), jnp.float32)
    z16 = jnp.zeros((STRIPE, CW), jnp.float32)
    ones = jnp.ones((CH, CW), jnp.float32)

    cnt_all = _make_sc_counts()(eidx, z16, ones)
    x0t = jnp.concatenate([_pack(xu0), _pack(xi0)], axis=0)
    s0 = _make_sc_agg()(eidx, x0t, z64)

    st = lambda a, b_: jnp.stack([a, b_], axis=0)
    b3 = lambda a, b_: jnp.stack([a.reshape(1, D), b_.reshape(1, D)], axis=0)
    wl0 = st(Wl[0, 0][_PERM, :], Wl[0, 1][_PERM, :])
    wr0 = st(Wr[0, 0] + Wr[0, 3], Wr[0, 1])
    bb0 = b3(bl[0, 0] + bl[0, 3], bl[0, 1])
    wl1 = st(Wl[1, 0][_PERM, :], Wl[1, 1][_PERM, :])
    wr1 = st(Wr[1, 0] + Wr[1, 3], Wr[1, 1])
    bb1 = b3(bl[1, 0] + bl[1, 3], bl[1, 1])
    wt_s = st(Wt[1], Wt[0])
    bt_s = b3(bt[1], bt[0])

    x0_all = st(xi0, xu0)
    x1_all, x1t = _tc_layer(s0, cnt_all, x0_all, wl0, wr0, bb0)
    s1 = _make_sc_agg()(eidx, x1t.reshape(NC * N_PAD, D), z64)
    f_all = _tc_final(s1, cnt_all, x1_all, x0_all, wl1, wr1, bb1, wt_s, bt_s)

    return (f_all[1, :N], f_all[0, :N])


# speedup vs baseline: 1.0023x; 1.0023x over previous
import functools

import jax
import jax.numpy as jnp
from jax import lax
from jax.experimental import pallas as pl
from jax.experimental.pallas import tpu as pltpu
from jax.experimental.pallas import tpu_sc as plsc

N = 25000
D = 64
E = 800000
NC, NS = 2, 16
CH = 80
TPB = 25
BLOCKS = 25
EPT = BLOCKS * TPB * CH
N_PAD = 25088
STRIPE = N_PAD // NS
CW = 16


def _mesh():
    return plsc.VectorSubcoreMesh(core_axis_name="c", subcore_axis_name="s",
                                  num_cores=NC, num_subcores=NS)


NBUF = 2
NFB = 2


def _sc_agg_body(eidx, xall, z64, s_all, idx_g, idx_s,
                 g0, g1, f0, f1, acc_sh, sg0, sg1, ss0, ss1,
                 sem_ig, sem_is):
    gbufs = (g0, g1)
    fbufs = (f0, f1)
    sgs = (sg0, sg1)
    sss = (ss0, ss1)
    c = lax.axis_index("c")
    s = lax.axis_index("s")
    row0 = s * STRIPE

    pltpu.sync_copy(z64, acc_sh.at[pl.ds(row0, STRIPE)])
    plsc.subcore_barrier()

    def fire_gather(j, par):
        pltpu.async_copy(xall.at[idx_g.at[par, j]], gbufs[j % NBUF],
                         sgs[j % NBUF])

    def wait_gather(j):
        pltpu.make_async_copy(xall.at[pl.ds(0, CH)], gbufs[j % NBUF],
                              sgs[j % NBUF]).wait()

    def convert(j):
        gb = gbufs[j % NBUF]
        fb = fbufs[j % NFB]

        @pl.loop(0, CH, step=8)
        def _rows(r):
            for rr in range(8):
                for k in range(2):
                    v = gb[r + rr, pl.ds(32 * k, 32)]
                    ev, od = plsc.unpack(v, format=plsc.PackFormat.INTERLEAVED)
                    fb[r + rr, pl.ds(16 * k, 16)] = ev
                    fb[r + rr, pl.ds(32 + 16 * k, 16)] = od

    pltpu.async_copy(eidx.at[c, s, 0], idx_g.at[0], sem_ig)
    pltpu.async_copy(eidx.at[2 + c, s, 0], idx_s.at[0], sem_is)

    @pl.loop(0, BLOCKS)
    def _block(b):
        par = lax.rem(b, 2)
        pltpu.make_async_copy(eidx.at[c, s, b], idx_g.at[par],
                              sem_ig).wait()
        pltpu.make_async_copy(eidx.at[2 + c, s, b], idx_s.at[par],
                              sem_is).wait()

        @pl.when(b + 1 < BLOCKS)
        def _():
            npar = lax.rem(b + 1, 2)
            pltpu.async_copy(eidx.at[c, s, b + 1], idx_g.at[npar], sem_ig)
            pltpu.async_copy(eidx.at[2 + c, s, b + 1], idx_s.at[npar],
                             sem_is)

        fire_gather(0, par)
        scat = [None] * TPB
        for j in range(TPB):
            if j + 1 < TPB:
                fire_gather(j + 1, par)
            wait_gather(j)
            if j >= NFB:
                scat[j - NFB].wait()
            convert(j)
            scat[j] = pltpu.async_copy(fbufs[j % NFB],
                                       acc_sh.at[idx_s.at[par, j]],
                                       sss[j % NFB], add=True)
        scat[TPB - 2].wait()
        scat[TPB - 1].wait()

    plsc.subcore_barrier()
    pltpu.sync_copy(acc_sh.at[pl.ds(row0, STRIPE)],
                    s_all.at[c, pl.ds(row0, STRIPE)])


def _sc_counts_body(eidx, z16, ones_h, cnt_all, idx_s, ones_v, cnt_sh,
                    sem_cs):
    c = lax.axis_index("c")
    s = lax.axis_index("s")
    row0 = s * STRIPE

    pltpu.sync_copy(z16, cnt_sh.at[pl.ds(row0, STRIPE)])
    pltpu.sync_copy(ones_h, ones_v)
    plsc.subcore_barrier()

    @pl.loop(0, BLOCKS)
    def _block(b):
        pltpu.sync_copy(eidx.at[2 + c, s, b], idx_s)
        scat = []
        for j in range(TPB):
            scat.append(pltpu.async_copy(ones_v, cnt_sh.at[idx_s.at[j]],
                                         sem_cs, add=True))
        for h in scat:
            h.wait()

    plsc.subcore_barrier()
    pltpu.sync_copy(cnt_sh.at[pl.ds(row0, STRIPE)],
                    cnt_all.at[c, pl.ds(row0, STRIPE)])


def _make_sc_agg():
    out = jax.ShapeDtypeStruct((NC, N_PAD, D), jnp.float32)
    scratch = [
        pltpu.VMEM((2, TPB, CH), jnp.int32),
        pltpu.VMEM((2, TPB, CH), jnp.int32),
    ] + [pltpu.VMEM((CH, D), jnp.bfloat16)] * NBUF \
      + [pltpu.VMEM((CH, D), jnp.float32)] * NFB + [
        pltpu.VMEM_SHARED((N_PAD, D), jnp.float32),
    ] + [pltpu.SemaphoreType.DMA] * (NBUF + NFB + 2)
    return pl.kernel(_sc_agg_body, out_type=out, mesh=_mesh(),
                     scratch_types=scratch,
                     compiler_params=pltpu.CompilerParams(
                         use_tc_tiling_on_sc=False,
                         needs_layout_passes=False))


def _make_sc_counts():
    out = jax.ShapeDtypeStruct((NC, N_PAD, CW), jnp.float32)
    scratch = [
        pltpu.VMEM((TPB, CH), jnp.int32),
        pltpu.VMEM((CH, CW), jnp.float32),
        pltpu.VMEM_SHARED((N_PAD, CW), jnp.float32),
        pltpu.SemaphoreType.DMA,
    ]
    return pl.kernel(_sc_counts_body, out_type=out, mesh=_mesh(),
                     scratch_types=scratch,
                     compiler_params=pltpu.CompilerParams(
                         use_tc_tiling_on_sc=False))


def _tc_layer_body(s_ref, c_ref, x_ref, wl_ref, wr_ref, b_ref,
                   o_ref, ot_ref):
    cnt = jnp.maximum(c_ref[0, :, 0:1], 1.0)
    mean = s_ref[0] / cnt
    acc = jnp.dot(mean, wl_ref[0], preferred_element_type=jnp.float32)
    acc += jnp.dot(x_ref[0], wr_ref[0], preferred_element_type=jnp.float32)
    o = jnp.maximum(acc + b_ref[0], 0.0)
    o_ref[0] = o
    ot_ref[0] = o.astype(jnp.bfloat16)


def _tc_final_body(s_ref, c_ref, x1_ref, x0_ref, wl_ref, wr_ref, b_ref,
                   wt_ref, bt_ref, o_ref):
    cnt = jnp.maximum(c_ref[0, :, 0:1], 1.0)
    mean = s_ref[0] / cnt
    acc = jnp.dot(mean, wl_ref[0], preferred_element_type=jnp.float32)
    acc += jnp.dot(x1_ref[0], wr_ref[0], preferred_element_type=jnp.float32)
    x2 = jnp.maximum(acc + b_ref[0], 0.0)
    m = (x0_ref[0] + x1_ref[0] + x2) * (1.0 / 3.0)
    o_ref[0] = jnp.dot(m, wt_ref[0],
                       preferred_element_type=jnp.float32) + bt_ref[0]


_RB = 1568


def _row_spec(w):
    return pl.BlockSpec((1, _RB, w), lambda d, i: (d, i, 0))


def _w_spec(r, w):
    return pl.BlockSpec((1, r, w), lambda d, i: (d, 0, 0))


def _tc_layer(S, cnt, x, wl, wr, b):
    return pl.pallas_call(
        _tc_layer_body,
        grid=(NC, N_PAD // _RB),
        in_specs=[_row_spec(D), _row_spec(CW), _row_spec(D),
                  _w_spec(D, D), _w_spec(D, D), _w_spec(1, D)],
        out_specs=[_row_spec(D),
                   pl.BlockSpec((1, _RB, D), lambda d, i: (1 - d, i, 0))],
        out_shape=[jax.ShapeDtypeStruct((NC, N_PAD, D), jnp.float32),
                   jax.ShapeDtypeStruct((NC, N_PAD, D), jnp.bfloat16)],
    )(S, cnt, x, wl, wr, b)


def _tc_final(S, cnt, x1, x0, wl, wr, b, wt, bt):
    return pl.pallas_call(
        _tc_final_body,
        grid=(NC, N_PAD // _RB),
        in_specs=[_row_spec(D), _row_spec(CW), _row_spec(D), _row_spec(D),
                  _w_spec(D, D), _w_spec(D, D), _w_spec(1, D),
                  _w_spec(D, D), _w_spec(1, D)],
        out_specs=_row_spec(D),
        out_shape=jax.ShapeDtypeStruct((NC, N_PAD, D), jnp.float32),
    )(S, cnt, x1, x0, wl, wr, b, wt, bt)


_PERM = tuple(
    [p for p in range(0, 32, 2)] + [p for p in range(32, 64, 2)]
    + [p for p in range(1, 32, 2)] + [p for p in range(33, 64, 2)])


def _pack(x):
    return x.astype(jnp.bfloat16)


def kernel(edge_index, emb_user, emb_item, emb_cat, Wl, bl, Wr, Wt, bt):
    del emb_cat
    edge_index = edge_index.astype(jnp.int32)

    eidx = jnp.stack([edge_index[0], edge_index[1] + N_PAD,
                      edge_index[1], edge_index[0]])
    eidx = eidx.reshape(4, NS, BLOCKS, TPB, CH)

    zpad = jnp.zeros((N_PAD - N, D), jnp.float32)
    xu0 = jnp.concatenate([emb_user, zpad], axis=0)
    xi0 = jnp.concatenate([emb_item, zpad], axis=0)

    z64 = jnp.zeros((STRIPE, D), jnp.float32)
    z16 = jnp.zeros((STRIPE, CW), jnp.float32)
    ones = jnp.ones((CH, CW), jnp.float32)

    cnt_all = _make_sc_counts()(eidx, z16, ones)
    x0t = jnp.concatenate([_pack(xu0), _pack(xi0)], axis=0)
    s0 = _make_sc_agg()(eidx, x0t, z64)

    st = lambda a, b_: jnp.stack([a, b_], axis=0)
    b3 = lambda a, b_: jnp.stack([a.reshape(1, D), b_.reshape(1, D)], axis=0)
    wl0 = st(Wl[0, 0][_PERM, :], Wl[0, 1][_PERM, :])
    wr0 = st(Wr[0, 0] + Wr[0, 3], Wr[0, 1])
    bb0 = b3(bl[0, 0] + bl[0, 3], bl[0, 1])
    wl1 = st(Wl[1, 0][_PERM, :], Wl[1, 1][_PERM, :])
    wr1 = st(Wr[1, 0] + Wr[1, 3], Wr[1, 1])
    bb1 = b3(bl[1, 0] + bl[1, 3], bl[1, 1])
    wt_s = st(Wt[1], Wt[0])
    bt_s = b3(bt[1], bt[0])

    x0_all = st(xi0, xu0)
    x1_all, x1t = _tc_layer(s0, cnt_all, x0_all, wl0, wr0, bb0)
    s1 = _make_sc_agg()(eidx, x1t.reshape(NC * N_PAD, D), z64)
    f_all = _tc_final(s1, cnt_all, x1_all, x0_all, wl1, wr1, bb1, wt_s, bt_s)

    return (f_all[1, :N], f_all[0, :N])
